# Initial kernel scaffold; baseline (speedup 1.0000x reference)
import jax
import jax.numpy as jnp
from jax.experimental import pallas as pl


def kernel(edge_index, node_feat, edge_feat, edge_embed, dim_size, W0, W1, W2, W_sc):
    raise NotImplementedError("write your pallas kernel here")


# speedup vs baseline: 2.3082x; 2.3082x over previous
import functools

import jax
import jax.numpy as jnp
import numpy as np
from jax import lax
from jax.experimental import pallas as pl
from jax.experimental.pallas import tpu as pltpu
from jax.experimental.pallas import tpu_sc as plsc

_ACT_CST = 1.6790

_NC = 2
_NS = 16
_LANES = 16


def _wf_tc(edge_embed, edge_feat, W0, W1, W2):
    E, R = edge_embed.shape
    H = W0.shape[1]
    D = W2.shape[1]
    BE = 3200
    assert E % BE == 0
    s0 = 1.0 / np.sqrt(W0.shape[0])
    s1 = 1.0 / np.sqrt(W1.shape[0])
    s2 = 1.0 / np.sqrt(W2.shape[0])

    def body(ee_ref, ef_ref, w0_ref, w1_ref, w2_ref, out_ref):
        x = ee_ref[...]
        h = jnp.dot(x, w0_ref[...] * s0, preferred_element_type=jnp.float32)
        h = jax.nn.silu(h) * _ACT_CST
        h = jnp.dot(h, w1_ref[...] * s1, preferred_element_type=jnp.float32)
        h = jax.nn.silu(h) * _ACT_CST
        w = jnp.dot(h, w2_ref[...] * s2, preferred_element_type=jnp.float32)
        out_ref[...] = w * ef_ref[...]

    return pl.pallas_call(
        body,
        grid=(E // BE,),
        in_specs=[
            pl.BlockSpec((BE, R), lambda i: (i, 0)),
            pl.BlockSpec((BE, 1), lambda i: (i, 0)),
            pl.BlockSpec((W0.shape[0], H), lambda i: (0, 0)),
            pl.BlockSpec((H, H), lambda i: (0, 0)),
            pl.BlockSpec((H, D), lambda i: (0, 0)),
        ],
        out_specs=pl.BlockSpec((BE, D), lambda i: (i, 0)),
        out_shape=jax.ShapeDtypeStruct((E, D), jnp.float32),
    )(edge_embed, edge_feat, W0, W1, W2)


def _gather_scatter_sc(src, dst, wf, node_feat, zeros):
    E = src.shape[0]
    N, D = node_feat.shape
    NW = _NC * _NS
    K = 80
    EPW = E // NW
    CH = EPW // K
    RPT = (N // _NS) & ~7
    TAIL = N - RPT * _NS
    assert EPW * NW == E and CH * K == EPW and TAIL >= 0 and TAIL % 8 == 0
    DJ = D // _LANES

    mesh = plsc.VectorSubcoreMesh(core_axis_name="c", subcore_axis_name="s")

    @functools.partial(
        pl.kernel,
        out_type=[jax.ShapeDtypeStruct((N, D), jnp.float32)] * 2,
        mesh=mesh,
        scratch_types=[
            pltpu.VMEM((K,), jnp.int32),
            pltpu.VMEM((K,), jnp.int32),
            pltpu.VMEM((K, D), jnp.float32),
            pltpu.VMEM((K, D), jnp.float32),
            pltpu.VMEM_SHARED((N, D), jnp.float32),
            pltpu.SemaphoreType.DMA,
        ],
    )
    def sc_kernel(src_hbm, dst_hbm, wf_hbm, node_hbm, zeros_hbm,
                  out0, out1, src_v, dst_v, rows_v, wf_v, acc, sem):
        c = lax.axis_index("c")
        s = lax.axis_index("s")
        pltpu.sync_copy(zeros_hbm.at[pl.ds(0, RPT)], acc.at[pl.ds(s * RPT, RPT)])
        if TAIL:
            @pl.when(s == 0)
            def _():
                pltpu.sync_copy(zeros_hbm.at[pl.ds(0, TAIL)],
                                acc.at[pl.ds(RPT * _NS, TAIL)])
        plsc.subcore_barrier()

        base0 = (c * _NS + s) * EPW

        def chunk(i, carry):
            base = base0 + i * K
            pltpu.sync_copy(src_hbm.at[pl.ds(base, K)], src_v)
            pltpu.sync_copy(dst_hbm.at[pl.ds(base, K)], dst_v)
            pltpu.async_copy(node_hbm.at[src_v], rows_v, sem).wait()
            pltpu.sync_copy(wf_hbm.at[pl.ds(base, K)], wf_v)

            def mul_body(e, carry2):
                for j in range(DJ):
                    sl = pl.ds(j * _LANES, _LANES)
                    wf_v[e, sl] = wf_v[e, sl] * rows_v[e, sl]
                return carry2

            lax.fori_loop(0, K, mul_body, 0)
            pltpu.sync_copy(wf_v, acc.at[dst_v], add=True)
            return carry

        lax.fori_loop(0, CH, chunk, 0)
        plsc.subcore_barrier()

        @pl.when(c == 0)
        def _():
            pltpu.sync_copy(acc.at[pl.ds(s * RPT, RPT)],
                            out0.at[pl.ds(s * RPT, RPT)])
            if TAIL:
                @pl.when(s == 0)
                def _():
                    pltpu.sync_copy(acc.at[pl.ds(RPT * _NS, TAIL)],
                                    out0.at[pl.ds(RPT * _NS, TAIL)])

        @pl.when(c == 1)
        def _():
            pltpu.sync_copy(acc.at[pl.ds(s * RPT, RPT)],
                            out1.at[pl.ds(s * RPT, RPT)])
            if TAIL:
                @pl.when(s == 0)
                def _():
                    pltpu.sync_copy(acc.at[pl.ds(RPT * _NS, TAIL)],
                                    out1.at[pl.ds(RPT * _NS, TAIL)])

    return sc_kernel(src, dst, wf, node_feat, zeros)


def _combine_tc(p0, p1, node_feat, W_sc):
    N, D = node_feat.shape
    BN = 2000
    assert N % BN == 0
    ssc = 1.0 / np.sqrt(W_sc.shape[0])

    def body(p0_ref, p1_ref, nf_ref, wsc_ref, out_ref):
        sc = jnp.dot(nf_ref[...], wsc_ref[...] * ssc,
                     preferred_element_type=jnp.float32)
        out_ref[...] = p0_ref[...] + p1_ref[...] + sc

    return pl.pallas_call(
        body,
        grid=(N // BN,),
        in_specs=[
            pl.BlockSpec((BN, D), lambda i: (i, 0)),
            pl.BlockSpec((BN, D), lambda i: (i, 0)),
            pl.BlockSpec((BN, D), lambda i: (i, 0)),
            pl.BlockSpec((D, D), lambda i: (0, 0)),
        ],
        out_specs=pl.BlockSpec((BN, D), lambda i: (i, 0)),
        out_shape=jax.ShapeDtypeStruct((N, D), jnp.float32),
    )(p0, p1, node_feat, W_sc)


def kernel(edge_index, node_feat, edge_feat, edge_embed, dim_size,
           W0, W1, W2, W_sc):
    src = edge_index[0]
    dst = edge_index[1]
    N, D = node_feat.shape
    wf = _wf_tc(edge_embed, edge_feat, W0, W1, W2)
    zeros = jnp.zeros((N // _NS, D), jnp.float32)
    p0, p1 = _gather_scatter_sc(src, dst, wf, node_feat, zeros)
    return _combine_tc(p0, p1, node_feat, W_sc)


# speedup vs baseline: 5.2790x; 2.2871x over previous
import functools

import jax
import jax.numpy as jnp
import numpy as np
from jax import lax
from jax.experimental import pallas as pl
from jax.experimental.pallas import tpu as pltpu
from jax.experimental.pallas import tpu_sc as plsc

_ACT_CST = 1.6790

_NC = 2
_NS = 16
_LANES = 16


def _wf_tc(edge_embed_t, edge_feat_t, W0, W1, W2):
    R, E = edge_embed_t.shape
    H = W0.shape[1]
    D = W2.shape[1]
    BE = 3200
    assert E % BE == 0
    s0 = 1.0 / np.sqrt(W0.shape[0])
    s1 = 1.0 / np.sqrt(W1.shape[0])
    s2 = 1.0 / np.sqrt(W2.shape[0])
    cdims = (((0,), (0,)), ((), ()))

    def body(ee_ref, ef_ref, w0_ref, w1_ref, w2_ref, out_ref):
        x = ee_ref[...]
        h = lax.dot_general(w0_ref[...] * s0, x, cdims,
                            preferred_element_type=jnp.float32)
        h = jax.nn.silu(h) * _ACT_CST
        h = lax.dot_general(w1_ref[...] * s1, h, cdims,
                            preferred_element_type=jnp.float32)
        h = jax.nn.silu(h) * _ACT_CST
        h = h * ef_ref[...]
        out_ref[...] = lax.dot_general(h, w2_ref[...] * s2, cdims,
                                       preferred_element_type=jnp.float32)

    return pl.pallas_call(
        body,
        grid=(E // BE,),
        in_specs=[
            pl.BlockSpec((R, BE), lambda i: (0, i)),
            pl.BlockSpec((1, BE), lambda i: (0, i)),
            pl.BlockSpec((W0.shape[0], H), lambda i: (0, 0)),
            pl.BlockSpec((H, H), lambda i: (0, 0)),
            pl.BlockSpec((H, D), lambda i: (0, 0)),
        ],
        out_specs=pl.BlockSpec((BE, D), lambda i: (i, 0)),
        out_shape=jax.ShapeDtypeStruct((E, D), jnp.float32),
    )(edge_embed_t, edge_feat_t, W0, W1, W2)


def _gather_scatter_sc(src3, dst3, wf, node_feat, zeros):
    NW = _NC * _NS
    _, CH, K = src3.shape
    E = NW * CH * K
    N, D = node_feat.shape
    RPT = (N // _NS) & ~7
    TAIL = N - RPT * _NS
    DJ = D // _LANES

    mesh = plsc.VectorSubcoreMesh(core_axis_name="c", subcore_axis_name="s")

    @functools.partial(
        pl.kernel,
        out_type=[jax.ShapeDtypeStruct((N, D), jnp.float32)] * 2,
        mesh=mesh,
        scratch_types=[
            [pltpu.VMEM((K,), jnp.int32)] * 4,
            [pltpu.VMEM((K,), jnp.int32)] * 4,
            [pltpu.VMEM((K, D), jnp.float32)] * 2,
            [pltpu.VMEM((K, D), jnp.float32)] * 2,
            pltpu.VMEM_SHARED((N, D), jnp.float32),
            [pltpu.SemaphoreType.DMA] * 4,
            [pltpu.SemaphoreType.DMA] * 2,
            [pltpu.SemaphoreType.DMA] * 2,
        ],
    )
    def sc_kernel(src_hbm, dst_hbm, wf_hbm, node_hbm, zeros_hbm,
                  out0, out1, srcb, dstb, rows, wfb, acc, isem, gsem, wsem):
        c = lax.axis_index("c")
        s = lax.axis_index("s")
        wid = c * _NS + s
        base0 = wid * (CH * K)

        pltpu.sync_copy(zeros_hbm.at[pl.ds(0, RPT)],
                        acc.at[pl.ds(s * RPT, RPT)])
        if TAIL:
            @pl.when(s == 0)
            def _():
                pltpu.sync_copy(zeros_hbm.at[pl.ds(0, TAIL)],
                                acc.at[pl.ds(RPT * _NS, TAIL)])
        plsc.subcore_barrier()

        def start_idx(i, q):
            pltpu.async_copy(src_hbm.at[wid, i], srcb[q], isem[q])
            pltpu.async_copy(dst_hbm.at[wid, i], dstb[q], isem[q])

        def wait_idx(i, q):
            pltpu.make_async_copy(src_hbm.at[wid, i], srcb[q], isem[q]).wait()
            pltpu.make_async_copy(dst_hbm.at[wid, i], dstb[q], isem[q]).wait()

        def start_data(i, p, q):
            pltpu.async_copy(node_hbm.at[srcb[q]], rows[p], gsem[p])
            pltpu.async_copy(wf_hbm.at[pl.ds(base0 + i * K, K)],
                             wfb[p], wsem[p])

        def wait_data(i, p, q):
            pltpu.make_async_copy(node_hbm.at[srcb[q]], rows[p],
                                  gsem[p]).wait()
            pltpu.make_async_copy(wf_hbm.at[pl.ds(base0 + i * K, K)],
                                  wfb[p], wsem[p]).wait()

        for q in range(4):
            start_idx(q, q)
        for p in range(2):
            wait_idx(p, p)
            start_data(p, p, p)

        def body(i, p, q):
            q2 = (q + 2) % 4
            wait_data(i, p, q)

            @plsc.parallel_loop(0, K, unroll=2)
            def _(e):
                for j in range(DJ):
                    sl = pl.ds(j * _LANES, _LANES)
                    wfb[p][e, sl] = wfb[p][e, sl] * rows[p][e, sl]

            pltpu.sync_copy(wfb[p], acc.at[dstb[q]], add=True)

            @pl.when(i + 2 < CH)
            def _():
                wait_idx(i + 2, q2)
                start_data(i + 2, p, q2)

            @pl.when(i + 4 < CH)
            def _():
                start_idx(i + 4, q)

        def chunk(i, carry):
            for p in range(2):
                for q in (p, p + 2):
                    @pl.when(lax.rem(i, 4) == q)
                    def _(i=i, p=p, q=q):
                        body(i, p, q)
            return carry

        lax.fori_loop(0, CH, chunk, 0)
        plsc.subcore_barrier()

        def drain(out):
            pltpu.sync_copy(acc.at[pl.ds(s * RPT, RPT)],
                            out.at[pl.ds(s * RPT, RPT)])
            if TAIL:
                @pl.when(s == 0)
                def _():
                    pltpu.sync_copy(acc.at[pl.ds(RPT * _NS, TAIL)],
                                    out.at[pl.ds(RPT * _NS, TAIL)])

        @pl.when(c == 0)
        def _():
            drain(out0)

        @pl.when(c == 1)
        def _():
            drain(out1)

    return sc_kernel(src3, dst3, wf, node_feat, zeros)


def _combine_tc(p0, p1, node_feat, W_sc):
    N, D = node_feat.shape
    BN = 2000
    assert N % BN == 0
    ssc = 1.0 / np.sqrt(W_sc.shape[0])

    def body(p0_ref, p1_ref, nf_ref, wsc_ref, out_ref):
        sc = jnp.dot(nf_ref[...], wsc_ref[...] * ssc,
                     preferred_element_type=jnp.float32)
        out_ref[...] = p0_ref[...] + p1_ref[...] + sc

    return pl.pallas_call(
        body,
        grid=(N // BN,),
        in_specs=[
            pl.BlockSpec((BN, D), lambda i: (i, 0)),
            pl.BlockSpec((BN, D), lambda i: (i, 0)),
            pl.BlockSpec((BN, D), lambda i: (i, 0)),
            pl.BlockSpec((D, D), lambda i: (0, 0)),
        ],
        out_specs=pl.BlockSpec((BN, D), lambda i: (i, 0)),
        out_shape=jax.ShapeDtypeStruct((N, D), jnp.float32),
    )(p0, p1, node_feat, W_sc)


def kernel(edge_index, node_feat, edge_feat, edge_embed, dim_size,
           W0, W1, W2, W_sc):
    N, D = node_feat.shape
    E = edge_index.shape[1]
    NW = _NC * _NS
    K = 40
    CH = E // (NW * K)
    src3 = edge_index[0].reshape(NW, CH, K)
    dst3 = edge_index[1].reshape(NW, CH, K)
    wf = _wf_tc(edge_embed.T, edge_feat.T, W0, W1, W2)
    zeros = jnp.zeros((N // _NS, D), jnp.float32)
    p0, p1 = _gather_scatter_sc(src3, dst3, wf, node_feat, zeros)
    return _combine_tc(p0, p1, node_feat, W_sc)
